# baseline (speedup 1.0000x reference)
import functools

import jax
import jax.numpy as jnp
from jax import lax
from jax.experimental import pallas as pl
from jax.experimental.pallas import tpu as pltpu
from jax.experimental.pallas import tpu_sc as plsc

_EMB = 128
_IDX_CHUNK = 128
_REPL = 1


def _sc_gather_left(table, idx):
    B = idx.shape[0]
    V = table.shape[0]
    info = plsc.get_sparse_core_info()
    nw = info.num_cores * info.num_subcores
    bpw = B // nw
    nchunks = bpw // _IDX_CHUNK
    mesh = plsc.VectorSubcoreMesh(core_axis_name="c", subcore_axis_name="s")

    @functools.partial(
        pl.kernel,
        mesh=mesh,
        out_type=jax.ShapeDtypeStruct((B, 2 * _EMB), jnp.float32),
        scratch_types=[
            pltpu.VMEM((bpw,), jnp.int32),
            pltpu.VMEM((bpw, _EMB), jnp.float32),
            pltpu.VMEM_SHARED((_REPL, V, _EMB), jnp.float32),
            pltpu.SemaphoreType.DMA,
            pltpu.SemaphoreType.DMA,
        ],
    )
    def k(table_hbm, idx_hbm, out_hbm, idx_v, rows_v, table_sp, sem, wsem):
        sid = lax.axis_index("s")
        wid = sid * info.num_cores + lax.axis_index("c")
        base = wid * bpw

        ns = info.num_subcores
        per_copy = ns // _REPL
        stripe = (-(-V // per_copy) + 7) // 8 * 8
        nfull = V // stripe
        rem = V - nfull * stripe
        for r in range(_REPL):
            for t in range(per_copy):
                if t < nfull:

                    @pl.when(sid == r * per_copy + t)
                    def _(r=r, t=t):
                        pltpu.sync_copy(
                            table_hbm.at[pl.ds(t * stripe, stripe)],
                            table_sp.at[r, pl.ds(t * stripe, stripe)],
                        )

                elif t == nfull and rem:

                    @pl.when(sid == r * per_copy + t)
                    def _(r=r, t=t):
                        pltpu.sync_copy(
                            table_hbm.at[pl.ds(nfull * stripe, rem)],
                            table_sp.at[r, pl.ds(nfull * stripe, rem)],
                        )

        pltpu.sync_copy(idx_hbm.at[pl.ds(base, bpw)], idx_v)
        plsc.subcore_barrier()
        my_copy = lax.rem(sid, _REPL)
        gathers = [
            pltpu.async_copy(
                table_sp.at[my_copy].at[idx_v.at[pl.ds(j * _IDX_CHUNK, _IDX_CHUNK)]],
                rows_v.at[pl.ds(j * _IDX_CHUNK, _IDX_CHUNK)],
                sem,
            )
            for j in range(nchunks)
        ]
        writes = []
        for j, g in enumerate(gathers):
            g.wait()
            writes.append(
                pltpu.async_copy(
                    rows_v.at[pl.ds(j * _IDX_CHUNK, _IDX_CHUNK)],
                    out_hbm.at[
                        pl.ds(base + j * _IDX_CHUNK, _IDX_CHUNK), pl.ds(0, _EMB)
                    ],
                    wsem,
                )
            )
        for w in writes:
            w.wait()

    return k(table, idx)


def _tc_genre_right(x, wt_ext, sc_out, bm=8192):
    B, C = x.shape

    def body(x_ref, wt_ref, sc_ref, out_ref):
        del sc_ref
        xf = x_ref[...].astype(jnp.float32)
        res = jnp.dot(
            xf, wt_ref[...], preferred_element_type=jnp.float32
        )
        out_ref[...] = res[:, :_EMB] / res[:, _EMB : _EMB + 1]

    return pl.pallas_call(
        body,
        grid=(B // bm,),
        in_specs=[
            pl.BlockSpec((bm, C), lambda i: (i, 0)),
            pl.BlockSpec((C, _EMB + 1), lambda i: (0, 0)),
            pl.BlockSpec(memory_space=pl.ANY),
        ],
        out_specs=pl.BlockSpec((bm, _EMB), lambda i: (i, 1)),
        out_shape=jax.ShapeDtypeStruct((B, 2 * _EMB), jnp.float32),
        input_output_aliases={2: 0},
    )(x, wt_ext, sc_out)


def kernel(x, embedding_rate, genre_weight):
    wt_pad = jnp.concatenate(
        [jnp.zeros((1, _EMB), jnp.float32), genre_weight.T], axis=0
    )
    mask_col = jnp.concatenate(
        [jnp.zeros((1, 1), jnp.float32), jnp.ones((x.shape[1] - 1, 1), jnp.float32)],
        axis=0,
    )
    wt_ext = jnp.concatenate([wt_pad, mask_col], axis=1)
    sc_out = _sc_gather_left(embedding_rate, x[:, 0])
    return _tc_genre_right(x, wt_ext, sc_out)
